# baseline (speedup 1.0000x reference)
import functools

import jax
import jax.numpy as jnp
from jax import lax
from jax.experimental import pallas as pl
from jax.experimental.pallas import tpu as pltpu
from jax.experimental.pallas import tpu_sc as plsc

N = 10000
E = 320000
D = 128

NC = 2
NS = 16
NW = NC * NS
CH = 128
NCHUNK = 80
SEG = 20
NSEG = NCHUNK // SEG
EPAD = NW * NCHUNK * CH
NPAD = 10112
STRIPE = NPAD // NS


LAST = N - (NS - 1) * STRIPE


def _sc_aggregate(x, zeros, srcs, dsts):
    mesh = plsc.VectorSubcoreMesh(core_axis_name="c", subcore_axis_name="s")

    @functools.partial(
        pl.kernel,
        out_type=jax.ShapeDtypeStruct((NC, NPAD, D), jnp.float32),
        mesh=mesh,
        scratch_types=[
            pltpu.VMEM((2, SEG, CH), jnp.int32),
            pltpu.VMEM((2, SEG, CH), jnp.int32),
            pltpu.VMEM((CH, D), jnp.float32),
            pltpu.VMEM((CH, D), jnp.float32),
            pltpu.VMEM_SHARED((NPAD, D), jnp.float32),
            pltpu.SemaphoreType.DMA,
            pltpu.SemaphoreType.DMA,
            pltpu.SemaphoreType.DMA,
            pltpu.SemaphoreType.DMA,
            pltpu.SemaphoreType.DMA,
        ],
    )
    def k(x_hbm, z_hbm, src_hbm, dst_hbm, out_hbm,
          src_v, dst_v, buf_a, buf_b, agg, sem_a, sem_b, sem_sa, sem_sb, sem_i):
        c = lax.axis_index("c")
        s = lax.axis_index("s")
        wid = c * NS + s
        stripe = pl.ds(s * STRIPE, STRIPE)
        @pl.when(jnp.logical_and(c == 0, s == NS - 1))
        def _():
            pltpu.sync_copy(x_hbm.at[pl.ds(s * STRIPE, LAST)],
                            agg.at[pl.ds(s * STRIPE, LAST)])

        @pl.when(jnp.logical_and(c == 0, s < NS - 1))
        def _():
            pltpu.sync_copy(x_hbm.at[stripe], agg.at[stripe])

        @pl.when(c == 1)
        def _():
            pltpu.sync_copy(z_hbm, agg.at[stripe])

        pltpu.async_copy(src_hbm.at[wid].at[0], src_v.at[0], sem_i)
        pltpu.async_copy(dst_hbm.at[wid].at[0], dst_v.at[0], sem_i)
        plsc.subcore_barrier()

        def wait_idx():
            pltpu.make_async_copy(src_hbm.at[0].at[0], src_v.at[0], sem_i).wait()
            pltpu.make_async_copy(dst_hbm.at[0].at[0], dst_v.at[0], sem_i).wait()

        def gather(src_row, buf, sem):
            pltpu.async_copy(x_hbm.at[src_row], buf, sem)

        def wait_gather(buf, sem):
            pltpu.make_async_copy(x_hbm.at[src_v.at[0].at[0]], buf, sem).wait()

        def wait_scatter(buf, sem):
            pltpu.make_async_copy(buf, agg.at[dst_v.at[0].at[0]], sem).wait()

        def segment(t, bank, next_bank):
            sv = src_v.at[bank]
            dv = dst_v.at[bank]
            wait_idx()

            @pl.when(t + 1 < NSEG)
            def _():
                pltpu.async_copy(src_hbm.at[wid].at[t + 1],
                                 src_v.at[next_bank], sem_i)
                pltpu.async_copy(dst_hbm.at[wid].at[t + 1],
                                 dst_v.at[next_bank], sem_i)

            gather(sv.at[0], buf_a, sem_a)
            gather(sv.at[1], buf_b, sem_b)

            @pl.loop(0, SEG // 2)
            def _(p):
                j = 2 * p
                wait_gather(buf_a, sem_a)
                pltpu.async_copy(buf_a, agg.at[dv.at[j]], sem_sa, add=True)
                wait_gather(buf_b, sem_b)
                pltpu.async_copy(buf_b, agg.at[dv.at[j + 1]], sem_sb, add=True)

                @pl.when(j + 2 < SEG)
                def _():
                    wait_scatter(buf_a, sem_sa)
                    gather(sv.at[j + 2], buf_a, sem_a)

                @pl.when(j + 3 < SEG)
                def _():
                    wait_scatter(buf_b, sem_sb)
                    gather(sv.at[j + 3], buf_b, sem_b)

            wait_scatter(buf_a, sem_sa)
            wait_scatter(buf_b, sem_sb)

        @pl.loop(0, NSEG // 2)
        def _(q):
            segment(2 * q, 0, 1)
            segment(2 * q + 1, 1, 0)

        plsc.subcore_barrier()
        pltpu.sync_copy(agg.at[stripe], out_hbm.at[c].at[stripe])

    return k(x, zeros, srcs, dsts)


def _tc_mlp(partials, W1, b1, g1, be1, W2, b2, g2, be2, g3, be3, Wf, bf):
    eps = 1e-5

    def body(p_ref, W1_ref, b1_ref, g1_ref, be1_ref, W2_ref, b2_ref, g2_ref,
             be2_ref, g3_ref, be3_ref, Wf_ref, bf_ref, out_ref):
        def bn(y, g, b):
            m = jnp.mean(y, axis=0, keepdims=True)
            v = jnp.mean((y - m) ** 2, axis=0, keepdims=True)
            return (y - m) * lax.rsqrt(v + eps) * g + b

        def mm_t(a, w):
            return lax.dot_general(a, w, (((1,), (1,)), ((), ())),
                                   precision=lax.Precision.HIGHEST,
                                   preferred_element_type=jnp.float32)

        h = p_ref[0, :N] + p_ref[1, :N]
        y = jnp.maximum(bn(mm_t(h, W1_ref[...]) + b1_ref[...],
                           g1_ref[...], be1_ref[...]), 0.0)
        y = bn(mm_t(y, W2_ref[...]) + b2_ref[...], g2_ref[...], be2_ref[...])
        y = jnp.maximum(bn(y, g3_ref[...], be3_ref[...]), 0.0)
        out_ref[...] = mm_t(y, Wf_ref[...]) + bf_ref[...]

    vecs = [v.reshape(1, -1) for v in (b1, g1, be1, b2, g2, be2, g3, be3, bf)]
    b1, g1, be1, b2, g2, be2, g3, be3, bf = vecs
    return pl.pallas_call(
        body,
        out_shape=jax.ShapeDtypeStruct((N, Wf.shape[0]), jnp.float32),
    )(partials, W1, b1, g1, be1, W2, b2, g2, be2, g3, be3, Wf, bf)


def kernel(x, edge_index, W1, b1, g1, be1, W2, b2, g2, be2, g3, be3, Wf, bf):
    pad = EPAD - E
    ar = jnp.arange(pad, dtype=jnp.int32)
    src = jnp.concatenate([edge_index[0].astype(jnp.int32), (ar * 37) % N])
    dst = jnp.concatenate([edge_index[1].astype(jnp.int32),
                           N + ar % (NPAD - N)])
    src = src.reshape(NW, NSEG, SEG, CH)
    dst = dst.reshape(NW, NSEG, SEG, CH)
    zeros = jnp.zeros((STRIPE, D), jnp.float32)
    partials = _sc_aggregate(x, zeros, src, dst)
    return _tc_mlp(partials, W1, b1, g1, be1, W2, b2, g2, be2, g3, be3, Wf, bf)


# speedup vs baseline: 1.2618x; 1.2618x over previous
import functools

import jax
import jax.numpy as jnp
from jax import lax
from jax.experimental import pallas as pl
from jax.experimental.pallas import tpu as pltpu
from jax.experimental.pallas import tpu_sc as plsc

N = 10000
E = 320000
D = 128

NC = 2
NS = 16
NW = NC * NS
CH = 128
NCHUNK = 80
SEG = 20
NSEG = NCHUNK // SEG
EPAD = NW * NCHUNK * CH
NPAD = 10112
STRIPE = NPAD // NS


def _sc_aggregate(x, xh, srcs, dsts):
    mesh = plsc.VectorSubcoreMesh(core_axis_name="c", subcore_axis_name="s")

    @functools.partial(
        pl.kernel,
        out_type=jax.ShapeDtypeStruct((NC, NPAD, D), jnp.float32),
        mesh=mesh,
        scratch_types=[
            pltpu.VMEM((2, SEG, CH), jnp.int32),
            pltpu.VMEM((2, SEG, CH), jnp.int32),
            pltpu.VMEM((CH, D), jnp.float32),
            pltpu.VMEM((CH, D), jnp.float32),
            pltpu.VMEM_SHARED((NPAD, D), jnp.float32),
            pltpu.SemaphoreType.DMA,
            pltpu.SemaphoreType.DMA,
            pltpu.SemaphoreType.DMA,
        ],
    )
    def k(x_hbm, xh_hbm, src_hbm, dst_hbm, out_hbm,
          src_v, dst_v, buf_a, buf_b, agg, sem_a, sem_b, sem_i):
        c = lax.axis_index("c")
        s = lax.axis_index("s")
        wid = c * NS + s
        stripe = pl.ds(s * STRIPE, STRIPE)
        pltpu.sync_copy(xh_hbm.at[stripe], agg.at[stripe])
        pltpu.async_copy(src_hbm.at[wid].at[0], src_v.at[0], sem_i)
        pltpu.async_copy(dst_hbm.at[wid].at[0], dst_v.at[0], sem_i)
        plsc.subcore_barrier()

        def wait_idx():
            pltpu.make_async_copy(src_hbm.at[0].at[0], src_v.at[0], sem_i).wait()
            pltpu.make_async_copy(dst_hbm.at[0].at[0], dst_v.at[0], sem_i).wait()

        def gather(src_row, buf, sem):
            pltpu.async_copy(x_hbm.at[src_row], buf, sem)

        def wait_gather(buf, sem):
            pltpu.make_async_copy(x_hbm.at[src_v.at[0].at[0]], buf, sem).wait()

        def segment(t, bank, next_bank):
            sv = src_v.at[bank]
            dv = dst_v.at[bank]
            wait_idx()

            @pl.when(t + 1 < NSEG)
            def _():
                pltpu.async_copy(src_hbm.at[wid].at[t + 1],
                                 src_v.at[next_bank], sem_i)
                pltpu.async_copy(dst_hbm.at[wid].at[t + 1],
                                 dst_v.at[next_bank], sem_i)

            gather(sv.at[0], buf_a, sem_a)
            gather(sv.at[1], buf_b, sem_b)

            @pl.loop(0, SEG // 2)
            def _(p):
                j = 2 * p
                wait_gather(buf_a, sem_a)
                pltpu.sync_copy(buf_a, agg.at[dv.at[j]], add=True)

                @pl.when(j + 2 < SEG)
                def _():
                    gather(sv.at[j + 2], buf_a, sem_a)

                wait_gather(buf_b, sem_b)
                pltpu.sync_copy(buf_b, agg.at[dv.at[j + 1]], add=True)

                @pl.when(j + 3 < SEG)
                def _():
                    gather(sv.at[j + 3], buf_b, sem_b)

        @pl.loop(0, NSEG // 2)
        def _(q):
            segment(2 * q, 0, 1)
            segment(2 * q + 1, 1, 0)

        plsc.subcore_barrier()
        pltpu.sync_copy(agg.at[stripe], out_hbm.at[c].at[stripe])

    return k(x, xh, srcs, dsts)


def _tc_mlp(partials, W1, b1, g1, be1, W2, b2, g2, be2, g3, be3, Wf, bf):
    eps = 1e-5

    def body(p_ref, W1_ref, b1_ref, g1_ref, be1_ref, W2_ref, b2_ref, g2_ref,
             be2_ref, g3_ref, be3_ref, Wf_ref, bf_ref, out_ref):
        def bn(y, g, b):
            m = jnp.mean(y, axis=0, keepdims=True)
            v = jnp.mean((y - m) ** 2, axis=0, keepdims=True)
            return (y - m) * lax.rsqrt(v + eps) * g + b

        def mm(a, wt):
            return jnp.dot(a, wt, preferred_element_type=jnp.float32)

        h = p_ref[0, :N] + p_ref[1, :N]
        y = jnp.maximum(bn(mm(h, W1_ref[...]) + b1_ref[...],
                           g1_ref[...], be1_ref[...]), 0.0)
        y = bn(mm(y, W2_ref[...]) + b2_ref[...], g2_ref[...], be2_ref[...])
        y = jnp.maximum(bn(y, g3_ref[...], be3_ref[...]), 0.0)
        out_ref[...] = mm(y, Wf_ref[...]) + bf_ref[...]

    vecs = [v.reshape(1, -1) for v in (b1, g1, be1, b2, g2, be2, g3, be3, bf)]
    b1, g1, be1, b2, g2, be2, g3, be3, bf = vecs
    W1, W2, Wf = W1.T, W2.T, Wf.T
    return pl.pallas_call(
        body,
        out_shape=jax.ShapeDtypeStruct((N, Wf.shape[0]), jnp.float32),
    )(partials, W1, b1, g1, be1, W2, b2, g2, be2, g3, be3, Wf, bf)


def kernel(x, edge_index, W1, b1, g1, be1, W2, b2, g2, be2, g3, be3, Wf, bf):
    pad = EPAD - E
    ar = jnp.arange(pad, dtype=jnp.int32)
    src = jnp.concatenate([edge_index[0].astype(jnp.int32), (ar * 37) % N])
    dst = jnp.concatenate([edge_index[1].astype(jnp.int32),
                           N + ar % (NPAD - N)])
    src = src.reshape(NW, NSEG, SEG, CH)
    dst = dst.reshape(NW, NSEG, SEG, CH)
    xh = jnp.pad(x * 0.5, ((0, NPAD - N), (0, 0)))
    partials = _sc_aggregate(x, xh, src, dst)
    return _tc_mlp(partials, W1, b1, g1, be1, W2, b2, g2, be2, g3, be3, Wf, bf)
